# baseline (speedup 1.0000x reference)
import functools

import jax
import jax.numpy as jnp
from jax.experimental import pallas as pl
from jax.experimental.pallas import tpu as pltpu


def _batch_body(P, PS, D, pat_ref, seq1_ref, seq2_ref, geo_ref, out_ref):
    b = pl.program_id(0)
    for i in range(P):
        p0 = pat_ref[(b * P + i) * 2 + 0]
        p1 = pat_ref[(b * P + i) * 2 + 1]
        r1 = seq1_ref[0, pl.ds(p0, PS), :]
        r2 = seq2_ref[0, pl.ds(p1, PS), :]
        g = geo_ref[0, i]
        blk = jnp.concatenate(
            [
                jnp.broadcast_to(r1[None, :, :], (PS, PS, D)),
                jnp.broadcast_to(r2[:, None, :], (PS, PS, D)),
                g[:, :, None],
            ],
            axis=2,
        )
        out_ref[0, i] = blk


def _make_blocks(seq1M, seq2M, patches_flat, geo, *, interpret=False):
    B, L, D = seq1M.shape
    _, P, PS, _ = geo.shape
    F = 2 * D + 1

    grid_spec = pltpu.PrefetchScalarGridSpec(
        num_scalar_prefetch=1,
        grid=(B,),
        in_specs=[
            pl.BlockSpec((1, L, D), lambda b, pat: (b, 0, 0)),
            pl.BlockSpec((1, L, D), lambda b, pat: (b, 0, 0)),
            pl.BlockSpec((1, P, PS, PS), lambda b, pat: (b, 0, 0, 0)),
        ],
        out_specs=pl.BlockSpec(
            (1, P, PS, PS, F), lambda b, pat: (b, 0, 0, 0, 0)
        ),
    )
    return pl.pallas_call(
        functools.partial(_batch_body, P, PS, D),
        grid_spec=grid_spec,
        out_shape=jax.ShapeDtypeStruct((B, P, PS, PS, F), jnp.float32),
        interpret=interpret,
    )(patches_flat, seq1M, seq2M, geo)


def kernel(seq1M, seq2M, patches, geo):
    B, P, _ = patches.shape
    patches_flat = patches.reshape(B * P * 2).astype(jnp.int32)
    return _make_blocks(seq1M, seq2M, patches_flat, geo)
